# baseline (speedup 1.0000x reference)
import functools

import numpy as np
import jax
import jax.numpy as jnp
from jax import lax
from jax.experimental import pallas as pl
from jax.experimental.pallas import tpu as pltpu
from jax.experimental.pallas import tpu_sc as plsc

NLVL = 8
PROBE = 16
NB = 65536
NC, NS = 2, 16
NW = NC * NS
NPW = NB // NW
PCH = 128
NCH = NPW // PCH
NG = PCH // 16
NROW = PCH * 8
NR = NROW // 128
NSTEP = NCH * NLVL


def _i32(v):
    return int(np.asarray([v], np.uint32).view(np.int32)[0])

C1Y, C1Z = _i32(2654435761), _i32(805459861)
C2Y, C2Z = _i32(2654435767), _i32(805459871)

_mesh = plsc.VectorSubcoreMesh(core_axis_name="c", subcore_axis_name="s")

NCB = 131072
CBW = NCB // NW
CBB = CBW // 128


@functools.partial(
    pl.kernel,
    out_type=jax.ShapeDtypeStruct((NCB, PROBE), jnp.float32),
    mesh=_mesh,
    compiler_params=pltpu.CompilerParams(
        needs_layout_passes=False, use_tc_tiling_on_sc=False),
    scratch_types=[
        pltpu.VMEM((2048,), jnp.float32),
        pltpu.VMEM((128, PROBE), jnp.float32),
        pltpu.SemaphoreType.DMA,
    ],
)
def _cb_softmax_sc(cv_h, out_h, raw, outb, sem):
    wid = lax.axis_index("s") * NC + lax.axis_index("c")
    iota = lax.iota(jnp.int32, 16)

    def block(k, _):
        kg = wid * CBB + k
        c0 = pltpu.make_async_copy(
            cv_h.at[pl.ds(kg * 1024, 1024)], raw.at[pl.ds(0, 1024)], sem)
        c1 = pltpu.make_async_copy(
            cv_h.at[pl.ds((1024 + kg) * 1024, 1024)], raw.at[pl.ds(1024, 1024)], sem)
        c0.start()
        c1.start()
        c0.wait()
        c1.wait()

        def jg_body(jg, _):
            es = []
            s = None
            for p in range(PROBE):
                v = raw[pl.ds(p * 128 + jg * 16, 16)]
                e = jnp.exp(v)
                es.append(e)
                s = e if p == 0 else s + e
            inv = 1.0 / s
            jvec = iota + jg * 16
            for p in range(PROBE):
                plsc.store_scatter(
                    outb, [jvec, jnp.full((16,), p, jnp.int32)], es[p] * inv)
            return 0

        lax.fori_loop(0, 8, jg_body, 0, unroll=False)
        pltpu.sync_copy(outb, out_h.at[pl.ds(kg * 128, 128)])
        return 0

    lax.fori_loop(0, CBB, block, 0, unroll=False)


@functools.partial(
    pl.kernel,
    out_type=jax.ShapeDtypeStruct((NB, 2 * NLVL), jnp.float32),
    mesh=_mesh,
    compiler_params=pltpu.CompilerParams(
        needs_layout_passes=False, use_tc_tiling_on_sc=False),
    scratch_types=[
        pltpu.VMEM((NPW,), jnp.float32),
        pltpu.VMEM((NPW,), jnp.float32),
        pltpu.VMEM((NPW,), jnp.float32),
        pltpu.VMEM((2 * PCH, 2 * NLVL), jnp.float32),
        pltpu.VMEM((2 * NROW,), jnp.int32),
        pltpu.VMEM((4 * NROW,), jnp.int32),
        pltpu.VMEM((2 * NROW, PROBE), jnp.float32),
        pltpu.VMEM((4 * NROW, PROBE), jnp.float32),
        pltpu.SemaphoreType.DMA,
        pltpu.SemaphoreType.DMA,
    ],
)
def _compact_hash_sc(xs_h, ys_h, zs_h, emb_h, cb_h, out_h, *rest):
    xv, yv, zv, ov, cbi, embi, cbr, embr, sem0, sem1 = rest

    wid = lax.axis_index("s") * NC + lax.axis_index("c")
    base = wid * NPW
    pltpu.sync_copy(xs_h.at[pl.ds(base, NPW)], xv)
    pltpu.sync_copy(ys_h.at[pl.ds(base, NPW)], yv)
    pltpu.sync_copy(zs_h.at[pl.ds(base, NPW)], zv)
    iota = lax.iota(jnp.int32, 16)

    def lvl_consts(lvl):
        resf = (jnp.int32(16) << lvl).astype(jnp.float32)
        rowmask = jnp.where(
            lvl == 0, 255,
            jnp.where(lvl == 1, 2047, jnp.where(lvl == 2, 16383, 32767)))
        rowoff = jnp.where(
            lvl == 0, 0,
            jnp.where(lvl == 1, 256,
                      jnp.where(lvl == 2, 2304, 18688 + (lvl - 3) * 32768)))
        return resf, rowmask, rowoff

    def gen(t, parity):
        lvl = t & (NLVL - 1)
        resf, rowmask, rowoff = lvl_consts(lvl)
        cboff = lvl * 16384
        col_ch = (t >> 3) * PCH
        d_c = parity * NROW
        d_e = parity * (2 * NROW)

        def gen_g(g, _):
            col0 = col_ch + g * 16
            x = xv[pl.ds(col0, 16)] * resf
            y = yv[pl.ds(col0, 16)] * resf
            z = zv[pl.ds(col0, 16)] * resf
            xi = x.astype(jnp.int32)
            yi = y.astype(jnp.int32)
            zi = z.astype(jnp.int32)
            x1 = xi + 1
            m1y0 = yi * C1Y
            m1y1 = m1y0 + C1Y
            m1z0 = zi * C1Z
            m1z1 = m1z0 + C1Z
            m2y0 = yi * C2Y
            m2y1 = m2y0 + C2Y
            m2z0 = zi * C2Z
            m2z1 = m2z0 + C2Z
            jb = g * 128
            for c in range(8):
                cx = x1 if c & 1 else xi
                a1 = cx ^ (m1y1 if c & 2 else m1y0) ^ (m1z1 if c & 4 else m1z0)
                a2 = cx ^ (m2y1 if c & 2 else m2y0) ^ (m2z1 if c & 4 else m2z0)
                h = (a1 & rowmask) + rowoff
                rx = (h << 1) - (h & 7)
                embi[pl.ds(d_e + jb + c * 16, 16)] = rx
                embi[pl.ds(d_e + NROW + jb + c * 16, 16)] = rx + 8
                cbi[pl.ds(d_c + jb + c * 16, 16)] = (a2 & 16383) + cboff
            return 0

        lax.fori_loop(0, NG, gen_g, 0, unroll=False)

    def copies_for(sem, pstat):
        cs = []
        for r in range(NR):
            sl = pl.ds(pstat * NROW + r * 128, 128)
            cs.append(pltpu.make_async_copy(cb_h.at[cbi.at[sl]], cbr.at[sl], sem))
        for r in range(2 * NR):
            sl = pl.ds(pstat * 2 * NROW + r * 128, 128)
            cs.append(pltpu.make_async_copy(emb_h.at[embi.at[sl]], embr.at[sl], sem))
        return cs

    def comb(t, parity):
        lvl = t & (NLVL - 1)
        resf, _, _ = lvl_consts(lvl)
        ch = t >> 3
        col_ch = ch * PCH
        chp = ch & 1
        d_c = parity * NROW
        d_e = parity * (2 * NROW)

        def comb_g(g, _):
            col0 = col_ch + g * 16
            x = xv[pl.ds(col0, 16)] * resf
            y = yv[pl.ds(col0, 16)] * resf
            z = zv[pl.ds(col0, 16)] * resf
            fx = x - x.astype(jnp.int32).astype(jnp.float32)
            fy = y - y.astype(jnp.int32).astype(jnp.float32)
            fz = z - z.astype(jnp.int32).astype(jnp.float32)
            wx = (1.0 - fx, fx)
            wy = (1.0 - fy, fy)
            wz = (1.0 - fz, fz)
            wxy = [wx[bx] * wy[by] for by in (0, 1) for bx in (0, 1)]
            jb = g * 128
            ax = None
            ay = None
            for c in range(8):
                rows = iota + (d_c + jb + c * 16)
                rowse = iota + (d_e + jb + c * 16)
                rowsy = iota + (d_e + NROW + jb + c * 16)
                accs = [None, None]
                accx = [None, None]
                accy = [None, None]
                for p in range(PROBE):
                    pv = jnp.full((16,), p, jnp.int32)
                    w = plsc.load_gather(cbr, [rows, pv])
                    e = jnp.exp(w)
                    ex = plsc.load_gather(embr, [rowse, pv])
                    ey = plsc.load_gather(embr, [rowsy, pv])
                    q = p & 1
                    accs[q] = e if p < 2 else accs[q] + e
                    accx[q] = e * ex if p < 2 else accx[q] + e * ex
                    accy[q] = e * ey if p < 2 else accy[q] + e * ey
                tw = wxy[c & 3] * wz[(c >> 2) & 1]
                twi = tw / (accs[0] + accs[1])
                fx = (accx[0] + accx[1]) * twi
                fy = (accy[0] + accy[1]) * twi
                ax = fx if c == 0 else ax + fx
                ay = fy if c == 0 else ay + fy
            rowsix = iota + (chp * PCH + g * 16)
            plsc.store_scatter(ov, [rowsix, jnp.full((16,), 2 * lvl, jnp.int32)], ax)
            plsc.store_scatter(ov, [rowsix, jnp.full((16,), 2 * lvl + 1, jnp.int32)], ay)
            return 0

        lax.fori_loop(0, NG, comb_g, 0, unroll=False)

        @pl.when(lvl == NLVL - 1)
        def _():
            pltpu.sync_copy(ov.at[pl.ds(chp * PCH, PCH)],
                            out_h.at[pl.ds(base + col_ch, PCH)])

    def step(t, _):
        parity = t & 1
        gen(t, parity)

        @pl.when(parity == 0)
        def _():
            for cp in copies_for(sem0, 0):
                cp.start()

        @pl.when(parity == 1)
        def _():
            for cp in copies_for(sem1, 1):
                cp.start()

        @pl.when(t > 0)
        def _():
            @pl.when(parity == 1)
            def _():
                for cp in copies_for(sem0, 0):
                    cp.wait()

            @pl.when(parity == 0)
            def _():
                for cp in copies_for(sem1, 1):
                    cp.wait()

            comb(t - 1, 1 - parity)

        return 0

    lax.fori_loop(0, NSTEP, step, 0, unroll=False)
    for cp in copies_for(sem1, 1):
        cp.wait()
    comb(NSTEP - 1, 1)


def kernel(inputs, embeddings, code_book):
    xs = inputs[:, 0]
    ys = inputs[:, 1]
    zs = inputs[:, 2]
    ev = embeddings.reshape(-1, 128, 2).transpose(0, 2, 1).reshape(-1, PROBE)
    return _compact_hash_sc(xs, ys, zs, ev, code_book)


# speedup vs baseline: 1.0137x; 1.0137x over previous
import functools

import numpy as np
import jax
import jax.numpy as jnp
from jax import lax
from jax.experimental import pallas as pl
from jax.experimental.pallas import tpu as pltpu
from jax.experimental.pallas import tpu_sc as plsc

NLVL = 8
PROBE = 16
NB = 65536
NC, NS = 2, 16
NW = NC * NS
NPW = NB // NW
PCH = 64
NCH = NPW // PCH
NG = PCH // 16
NROW = PCH * 8
NR = NROW // 128
NSTEP = NCH * NLVL


def _i32(v):
    return int(np.asarray([v], np.uint32).view(np.int32)[0])

C1Y, C1Z = _i32(2654435761), _i32(805459861)
C2Y, C2Z = _i32(2654435767), _i32(805459871)

_mesh = plsc.VectorSubcoreMesh(core_axis_name="c", subcore_axis_name="s")


@functools.partial(
    pl.kernel,
    out_type=jax.ShapeDtypeStruct((NB, 2 * NLVL), jnp.float32),
    mesh=_mesh,
    compiler_params=pltpu.CompilerParams(
        needs_layout_passes=False, use_tc_tiling_on_sc=False),
    scratch_types=[
        pltpu.VMEM((NPW,), jnp.float32),
        pltpu.VMEM((NPW,), jnp.float32),
        pltpu.VMEM((NPW,), jnp.float32),
        pltpu.VMEM((NPW, 2 * NLVL), jnp.float32),
        pltpu.VMEM((2 * NROW,), jnp.int32),
        pltpu.VMEM((4 * NROW,), jnp.int32),
        pltpu.VMEM((2 * NROW, PROBE), jnp.float32),
        pltpu.VMEM((4 * NROW, PROBE), jnp.float32),
        pltpu.SemaphoreType.DMA,
        pltpu.SemaphoreType.DMA,
    ],
)
def _compact_hash_sc(xs_h, ys_h, zs_h, emb_h, cb_h, out_h, *rest):
    xv, yv, zv, ov, cbi, embi, cbr, embr, sem0, sem1 = rest

    wid = lax.axis_index("s") * NC + lax.axis_index("c")
    base = wid * NPW
    pltpu.sync_copy(xs_h.at[pl.ds(base, NPW)], xv)
    pltpu.sync_copy(ys_h.at[pl.ds(base, NPW)], yv)
    pltpu.sync_copy(zs_h.at[pl.ds(base, NPW)], zv)
    iota = lax.iota(jnp.int32, 16)

    def lvl_consts(lvl):
        resf = (jnp.int32(16) << lvl).astype(jnp.float32)
        rowmask = jnp.where(
            lvl == 0, 255,
            jnp.where(lvl == 1, 2047, jnp.where(lvl == 2, 16383, 32767)))
        rowoff = jnp.where(
            lvl == 0, 0,
            jnp.where(lvl == 1, 256,
                      jnp.where(lvl == 2, 2304, 18688 + (lvl - 3) * 32768)))
        return resf, rowmask, rowoff

    def gen(t, parity):
        lvl = t & (NLVL - 1)
        resf, rowmask, rowoff = lvl_consts(lvl)
        cboff = lvl * 16384
        col_ch = (t >> 3) * PCH
        d_c = parity * NROW
        d_e = parity * (2 * NROW)

        def gen_g(g, _):
            col0 = col_ch + g * 16
            x = xv[pl.ds(col0, 16)] * resf
            y = yv[pl.ds(col0, 16)] * resf
            z = zv[pl.ds(col0, 16)] * resf
            xi = x.astype(jnp.int32)
            yi = y.astype(jnp.int32)
            zi = z.astype(jnp.int32)
            x1 = xi + 1
            m1y0 = yi * C1Y
            m1y1 = m1y0 + C1Y
            m1z0 = zi * C1Z
            m1z1 = m1z0 + C1Z
            m2y0 = yi * C2Y
            m2y1 = m2y0 + C2Y
            m2z0 = zi * C2Z
            m2z1 = m2z0 + C2Z
            jb = g * 128
            for c in range(8):
                cx = x1 if c & 1 else xi
                a1 = cx ^ (m1y1 if c & 2 else m1y0) ^ (m1z1 if c & 4 else m1z0)
                a2 = cx ^ (m2y1 if c & 2 else m2y0) ^ (m2z1 if c & 4 else m2z0)
                h = (a1 & rowmask) + rowoff
                rx = (h << 1) - (h & 7)
                embi[pl.ds(d_e + jb + c * 16, 16)] = rx
                embi[pl.ds(d_e + NROW + jb + c * 16, 16)] = rx + 8
                cbi[pl.ds(d_c + jb + c * 16, 16)] = (a2 & 16383) + cboff
            return 0

        lax.fori_loop(0, NG, gen_g, 0, unroll=False)

    def copies_for(sem, pstat):
        cs = []
        for r in range(NR):
            sl = pl.ds(pstat * NROW + r * 128, 128)
            cs.append(pltpu.make_async_copy(cb_h.at[cbi.at[sl]], cbr.at[sl], sem))
        for r in range(2 * NR):
            sl = pl.ds(pstat * 2 * NROW + r * 128, 128)
            cs.append(pltpu.make_async_copy(emb_h.at[embi.at[sl]], embr.at[sl], sem))
        return cs

    def comb(t, parity):
        lvl = t & (NLVL - 1)
        resf, _, _ = lvl_consts(lvl)
        col_ch = (t >> 3) * PCH
        d_c = parity * NROW
        d_e = parity * (2 * NROW)

        def comb_g(g, _):
            col0 = col_ch + g * 16
            x = xv[pl.ds(col0, 16)] * resf
            y = yv[pl.ds(col0, 16)] * resf
            z = zv[pl.ds(col0, 16)] * resf
            fx = x - x.astype(jnp.int32).astype(jnp.float32)
            fy = y - y.astype(jnp.int32).astype(jnp.float32)
            fz = z - z.astype(jnp.int32).astype(jnp.float32)
            wx = (1.0 - fx, fx)
            wy = (1.0 - fy, fy)
            wz = (1.0 - fz, fz)
            wxy = [wx[bx] * wy[by] for by in (0, 1) for bx in (0, 1)]
            jb = g * 128
            ax = None
            ay = None
            for c in range(8):
                rows = iota + (d_c + jb + c * 16)
                rowse = iota + (d_e + jb + c * 16)
                rowsy = iota + (d_e + NROW + jb + c * 16)
                accs = [None, None]
                accx = [None, None]
                accy = [None, None]
                for p in range(PROBE):
                    pv = jnp.full((16,), p, jnp.int32)
                    w = plsc.load_gather(cbr, [rows, pv])
                    e = jnp.exp(w)
                    ex = plsc.load_gather(embr, [rowse, pv])
                    ey = plsc.load_gather(embr, [rowsy, pv])
                    q = p & 1
                    accs[q] = e if p < 2 else accs[q] + e
                    accx[q] = e * ex if p < 2 else accx[q] + e * ex
                    accy[q] = e * ey if p < 2 else accy[q] + e * ey
                tw = wxy[c & 3] * wz[(c >> 2) & 1]
                twi = tw / (accs[0] + accs[1])
                fx = (accx[0] + accx[1]) * twi
                fy = (accy[0] + accy[1]) * twi
                ax = fx if c == 0 else ax + fx
                ay = fy if c == 0 else ay + fy
            rowsix = iota + col0
            plsc.store_scatter(ov, [rowsix, jnp.full((16,), 2 * lvl, jnp.int32)], ax)
            plsc.store_scatter(ov, [rowsix, jnp.full((16,), 2 * lvl + 1, jnp.int32)], ay)
            return 0

        lax.fori_loop(0, NG, comb_g, 0, unroll=False)

    def step(t, _):
        parity = t & 1
        gen(t, parity)

        @pl.when(parity == 0)
        def _():
            for cp in copies_for(sem0, 0):
                cp.start()

        @pl.when(parity == 1)
        def _():
            for cp in copies_for(sem1, 1):
                cp.start()

        @pl.when(t > 0)
        def _():
            @pl.when(parity == 1)
            def _():
                for cp in copies_for(sem0, 0):
                    cp.wait()

            @pl.when(parity == 0)
            def _():
                for cp in copies_for(sem1, 1):
                    cp.wait()

            comb(t - 1, 1 - parity)

        return 0

    lax.fori_loop(0, NSTEP, step, 0, unroll=False)
    for cp in copies_for(sem1, 1):
        cp.wait()
    comb(NSTEP - 1, 1)

    pltpu.sync_copy(ov, out_h.at[pl.ds(base, NPW)])


def kernel(inputs, embeddings, code_book):
    xs = inputs[:, 0]
    ys = inputs[:, 1]
    zs = inputs[:, 2]
    ev = embeddings.reshape(-1, 128, 2).transpose(0, 2, 1).reshape(-1, PROBE)
    return _compact_hash_sc(xs, ys, zs, ev, code_book)
